# Initial kernel scaffold; baseline (speedup 1.0000x reference)
import functools

import jax
import jax.numpy as jnp
import numpy as np
from jax import lax
from jax.experimental import pallas as pl
from jax.experimental.pallas import tpu as pltpu
from jax.experimental.pallas import tpu_sc as plsc

EPS = 1e-5
NC = 2
NS = 16
NW = NC * NS


def _chunk(ept):
    for k in range(128, 0, -8):
        if ept % k == 0:
            return k
    return None


def _sc_degree(dst, zeros1d, n):
    e = dst.shape[0]
    ept = e // NW
    k = _chunk(ept)
    niter = ept // k
    mesh = plsc.VectorSubcoreMesh(core_axis_name="c", subcore_axis_name="s")

    @functools.partial(
        pl.kernel, mesh=mesh,
        out_type=jax.ShapeDtypeStruct((NC, n), jnp.float32),
        scratch_types=[
            pltpu.VMEM((k,), jnp.int32),
            pltpu.VMEM((k,), jnp.float32),
            pltpu.VMEM_SHARED((n,), jnp.float32),
        ],
    )
    def deg_kernel(dst_hbm, z_hbm, out_hbm, didx, ones_v, acc):
        cid = lax.axis_index("c")
        sid = lax.axis_index("s")
        wid = sid * NC + cid
        for j in range(k // 16):
            ones_v[pl.ds(16 * j, 16)] = jnp.ones((16,), jnp.float32)

        @pl.when(sid == 0)
        def _():
            pltpu.sync_copy(z_hbm, acc)

        plsc.subcore_barrier()

        def body(i, _):
            base = wid * ept + i * k
            pltpu.sync_copy(dst_hbm.at[pl.ds(base, k)], didx)
            pltpu.sync_copy(ones_v, acc.at[didx], add=True)
            return ()

        lax.fori_loop(0, niter, body, (), unroll=False)
        plsc.subcore_barrier()

        @pl.when(sid == 0)
        def _():
            pltpu.sync_copy(acc, out_hbm.at[cid])

    return deg_kernel(dst, zeros1d)


def _sc_agg(src, dst, hw, zeros2d, n, d):
    e = src.shape[0]
    ept = e // NW
    k = _chunk(ept)
    niter = ept // k
    rpt = n // NS
    mesh = plsc.VectorSubcoreMesh(core_axis_name="c", subcore_axis_name="s")

    @functools.partial(
        pl.kernel, mesh=mesh,
        out_type=jax.ShapeDtypeStruct((NC, n, d), jnp.float32),
        scratch_types=[
            pltpu.VMEM((k,), jnp.int32),
            pltpu.VMEM((k,), jnp.int32),
            pltpu.VMEM((k, d), jnp.float32),
            pltpu.VMEM_SHARED((n, d), jnp.float32),
            pltpu.SemaphoreType.DMA,
        ],
    )
    def agg_kernel(src_hbm, dst_hbm, hw_hbm, z_hbm, out_hbm,
                   sidx, didx, rows, acc, sem):
        cid = lax.axis_index("c")
        sid = lax.axis_index("s")
        wid = sid * NC + cid

        pltpu.sync_copy(z_hbm.at[pl.ds(sid * rpt, rpt)],
                        acc.at[pl.ds(sid * rpt, rpt)])
        plsc.subcore_barrier()

        def body(i, _):
            base = wid * ept + i * k
            pltpu.sync_copy(src_hbm.at[pl.ds(base, k)], sidx)
            pltpu.sync_copy(dst_hbm.at[pl.ds(base, k)], didx)
            pltpu.async_copy(hw_hbm.at[sidx], rows, sem).wait()
            pltpu.sync_copy(rows, acc.at[didx], add=True)
            return ()

        lax.fori_loop(0, niter, body, (), unroll=False)
        plsc.subcore_barrier()
        pltpu.sync_copy(acc.at[pl.ds(sid * rpt, rpt)],
                        out_hbm.at[cid, pl.ds(sid * rpt, rpt)])

    return agg_kernel(src, dst, hw, zeros2d)


_BN = float(1.0 / np.sqrt(1.0 + EPS))


def _tc_pre(x, W, degT, n, d, blk):
    def body(x_ref, w_ref, deg_ref, hw_ref, dinv_ref):
        dg = deg_ref[:, 0:1] + deg_ref[:, 1:2] + 1.0
        dinv = lax.rsqrt(dg)
        dinv_ref[...] = dinv
        hw_ref[...] = jnp.dot(x_ref[...], w_ref[...],
                              preferred_element_type=jnp.float32) * dinv

    grid = (n // blk,)
    return pl.pallas_call(
        body,
        grid=grid,
        in_specs=[
            pl.BlockSpec((blk, d), lambda i: (i, 0)),
            pl.BlockSpec((d, d), lambda i: (0, 0)),
            pl.BlockSpec((blk, 2), lambda i: (i, 0)),
        ],
        out_specs=[
            pl.BlockSpec((blk, d), lambda i: (i, 0)),
            pl.BlockSpec((blk, 1), lambda i: (i, 0)),
        ],
        out_shape=[
            jax.ShapeDtypeStruct((n, d), jnp.float32),
            jax.ShapeDtypeStruct((n, 1), jnp.float32),
        ],
    )(x, W, degT)


def _tc_layer(a0, a1, hws, dinv, b, g, bt, Wn, n, d, blk):
    def body(a0_ref, a1_ref, hws_ref, dinv_ref, b_ref, g_ref, bt_ref, w_ref,
             h_ref, hwn_ref):
        dinv = dinv_ref[...]
        pre = (a0_ref[...] + a1_ref[...] + hws_ref[...]) * dinv + b_ref[...]
        t = pre * (g_ref[...] * _BN) + bt_ref[...]
        h = jnp.maximum(t, 0.0)
        h_ref[...] = h
        hwn_ref[...] = jnp.dot(h, w_ref[...],
                               preferred_element_type=jnp.float32) * dinv

    grid = (n // blk,)
    return pl.pallas_call(
        body,
        grid=grid,
        in_specs=[
            pl.BlockSpec((blk, d), lambda i: (i, 0)),
            pl.BlockSpec((blk, d), lambda i: (i, 0)),
            pl.BlockSpec((blk, d), lambda i: (i, 0)),
            pl.BlockSpec((blk, 1), lambda i: (i, 0)),
            pl.BlockSpec((1, d), lambda i: (0, 0)),
            pl.BlockSpec((1, d), lambda i: (0, 0)),
            pl.BlockSpec((1, d), lambda i: (0, 0)),
            pl.BlockSpec((d, d), lambda i: (0, 0)),
        ],
        out_specs=[
            pl.BlockSpec((blk, d), lambda i: (i, 0)),
            pl.BlockSpec((blk, d), lambda i: (i, 0)),
        ],
        out_shape=[
            jax.ShapeDtypeStruct((n, d), jnp.float32),
            jax.ShapeDtypeStruct((n, d), jnp.float32),
        ],
    )(a0, a1, hws, dinv, b, g, bt, Wn)


def _tc_final(a0, a1, hws, dinv, b, g, bt, x, h1, h2, Wc, bc, n, d, c, blk):
    def body(a0_ref, a1_ref, hws_ref, dinv_ref, b_ref, g_ref, bt_ref,
             x_ref, h1_ref, h2_ref, wc_ref, bc_ref, out_ref):
        dinv = dinv_ref[...]
        pre = (a0_ref[...] + a1_ref[...] + hws_ref[...]) * dinv + b_ref[...]
        t = pre * (g_ref[...] * _BN) + bt_ref[...]
        h3 = jnp.maximum(t, 0.0)
        wc = wc_ref[...]
        logits = (
            jnp.dot(x_ref[...], wc[0 * d:1 * d], preferred_element_type=jnp.float32)
            + jnp.dot(h1_ref[...], wc[1 * d:2 * d], preferred_element_type=jnp.float32)
            + jnp.dot(h2_ref[...], wc[2 * d:3 * d], preferred_element_type=jnp.float32)
            + jnp.dot(h3, wc[3 * d:4 * d], preferred_element_type=jnp.float32)
            + bc_ref[...]
        )
        m = jnp.max(logits, axis=-1, keepdims=True)
        ex = jnp.exp(logits - m)
        lse = jnp.log(jnp.sum(ex, axis=-1, keepdims=True)) + m
        out_ref[...] = logits - lse

    grid = (n // blk,)
    return pl.pallas_call(
        body,
        grid=grid,
        in_specs=[
            pl.BlockSpec((blk, d), lambda i: (i, 0)),
            pl.BlockSpec((blk, d), lambda i: (i, 0)),
            pl.BlockSpec((blk, d), lambda i: (i, 0)),
            pl.BlockSpec((blk, 1), lambda i: (i, 0)),
            pl.BlockSpec((1, d), lambda i: (0, 0)),
            pl.BlockSpec((1, d), lambda i: (0, 0)),
            pl.BlockSpec((1, d), lambda i: (0, 0)),
            pl.BlockSpec((blk, d), lambda i: (i, 0)),
            pl.BlockSpec((blk, d), lambda i: (i, 0)),
            pl.BlockSpec((blk, d), lambda i: (i, 0)),
            pl.BlockSpec((4 * d, c), lambda i: (0, 0)),
            pl.BlockSpec((1, c), lambda i: (0, 0)),
        ],
        out_specs=pl.BlockSpec((blk, c), lambda i: (i, 0)),
        out_shape=jax.ShapeDtypeStruct((n, c), jnp.float32),
    )(a0, a1, hws, dinv, b, g, bt, x, h1, h2, Wc, bc)


def kernel(x, edge_index, W1, b1, g1, bt1, W2, b2, g2, bt2,
           W3, b3, g3, bt3, Wc, bc):
    n, d = x.shape
    c = Wc.shape[1]
    blk = 1000 if n % 1000 == 0 else 8
    src = edge_index[0]
    dst = edge_index[1]
    zeros1d = jnp.zeros((n,), jnp.float32)
    zeros2d = jnp.zeros((n, d), jnp.float32)

    degp = _sc_degree(dst, zeros1d, n)
    degT = degp.T
    b1r, g1r, bt1r = b1[None, :], g1[None, :], bt1[None, :]
    b2r, g2r, bt2r = b2[None, :], g2[None, :], bt2[None, :]
    b3r, g3r, bt3r = b3[None, :], g3[None, :], bt3[None, :]
    bcr = bc[None, :]

    hw1, dinv = _tc_pre(x, W1, degT, n, d, blk)
    a1 = _sc_agg(src, dst, hw1, zeros2d, n, d)
    h1, hw2 = _tc_layer(a1[0], a1[1], hw1, dinv, b1r, g1r, bt1r, W2, n, d, blk)
    a2 = _sc_agg(src, dst, hw2, zeros2d, n, d)
    h2, hw3 = _tc_layer(a2[0], a2[1], hw2, dinv, b2r, g2r, bt2r, W3, n, d, blk)
    a3 = _sc_agg(src, dst, hw3, zeros2d, n, d)
    return _tc_final(a3[0], a3[1], hw3, dinv, b3r, g3r, bt3r,
                     x, h1, h2, Wc, bcr, n, d, c, blk)


# speedup vs baseline: 10.1997x; 10.1997x over previous
import functools

import jax
import jax.numpy as jnp
import numpy as np
from jax import lax
from jax.experimental import pallas as pl
from jax.experimental.pallas import tpu as pltpu
from jax.experimental.pallas import tpu_sc as plsc

EPS = 1e-5
NC = 2
NS = 16
NW = NC * NS


def _chunk(ept):
    for k in range(128, 0, -8):
        if ept % k == 0:
            return k
    return None


def _sc_degree(dst, zeros1d, n):
    e = dst.shape[0]
    ept = e // NW
    k = _chunk(ept)
    niter = ept // k
    mesh = plsc.VectorSubcoreMesh(core_axis_name="c", subcore_axis_name="s")

    @functools.partial(
        pl.kernel, mesh=mesh,
        out_type=jax.ShapeDtypeStruct((NC, n), jnp.float32),
        scratch_types=[
            pltpu.VMEM((k,), jnp.int32),
            pltpu.VMEM((k,), jnp.float32),
            pltpu.VMEM_SHARED((n,), jnp.float32),
        ],
    )
    def deg_kernel(dst_hbm, z_hbm, out_hbm, didx, ones_v, acc):
        cid = lax.axis_index("c")
        sid = lax.axis_index("s")
        wid = sid * NC + cid
        for j in range(k // 16):
            ones_v[pl.ds(16 * j, 16)] = jnp.ones((16,), jnp.float32)

        @pl.when(sid == 0)
        def _():
            pltpu.sync_copy(z_hbm, acc)

        plsc.subcore_barrier()

        def body(i, _):
            base = wid * ept + i * k
            pltpu.sync_copy(dst_hbm.at[pl.ds(base, k)], didx)
            pltpu.sync_copy(ones_v, acc.at[didx], add=True)
            return ()

        lax.fori_loop(0, niter, body, (), unroll=False)
        plsc.subcore_barrier()

        @pl.when(sid == 0)
        def _():
            pltpu.sync_copy(acc, out_hbm.at[cid])

    return deg_kernel(dst, zeros1d)


def _sc_agg(src, dst, hw, zeros2d, n, d):
    e = src.shape[0]
    ept = e // NW
    k = _chunk(ept)
    niter = ept // k
    rpt = (n // NS) // 8 * 8
    tail = n - NS * rpt
    mesh = plsc.VectorSubcoreMesh(core_axis_name="c", subcore_axis_name="s")

    @functools.partial(
        pl.kernel, mesh=mesh,
        out_type=jax.ShapeDtypeStruct((NC, n, d), jnp.float32),
        scratch_types=[
            pltpu.VMEM((k,), jnp.int32),
            pltpu.VMEM((k,), jnp.int32),
            pltpu.VMEM((k, d), jnp.float32),
            pltpu.VMEM_SHARED((n, d), jnp.float32),
            pltpu.SemaphoreType.DMA,
        ],
    )
    def agg_kernel(src_hbm, dst_hbm, hw_hbm, z_hbm, out_hbm,
                   sidx, didx, rows, acc, sem):
        cid = lax.axis_index("c")
        sid = lax.axis_index("s")
        wid = sid * NC + cid

        pltpu.sync_copy(z_hbm.at[pl.ds(sid * rpt, rpt)],
                        acc.at[pl.ds(sid * rpt, rpt)])
        if tail:
            @pl.when(sid == NS - 1)
            def _():
                pltpu.sync_copy(z_hbm.at[pl.ds(NS * rpt, tail)],
                                acc.at[pl.ds(NS * rpt, tail)])
        plsc.subcore_barrier()

        def body(i, _):
            base = wid * ept + i * k
            pltpu.sync_copy(src_hbm.at[pl.ds(base, k)], sidx)
            pltpu.sync_copy(dst_hbm.at[pl.ds(base, k)], didx)
            pltpu.async_copy(hw_hbm.at[sidx], rows, sem).wait()
            pltpu.sync_copy(rows, acc.at[didx], add=True)
            return ()

        lax.fori_loop(0, niter, body, (), unroll=False)
        plsc.subcore_barrier()
        pltpu.sync_copy(acc.at[pl.ds(sid * rpt, rpt)],
                        out_hbm.at[cid, pl.ds(sid * rpt, rpt)])
        if tail:
            @pl.when(sid == NS - 1)
            def _():
                pltpu.sync_copy(acc.at[pl.ds(NS * rpt, tail)],
                                out_hbm.at[cid, pl.ds(NS * rpt, tail)])

    return agg_kernel(src, dst, hw, zeros2d)


_BN = float(1.0 / np.sqrt(1.0 + EPS))


def _tc_pre(x, W, degT, n, d, blk):
    def body(x_ref, w_ref, deg_ref, hw_ref, dinv_ref):
        dg = deg_ref[:, 0:1] + deg_ref[:, 1:2] + 1.0
        dinv = lax.rsqrt(dg)
        dinv_ref[...] = dinv
        hw_ref[...] = jnp.dot(x_ref[...], w_ref[...],
                              preferred_element_type=jnp.float32) * dinv

    grid = (n // blk,)
    return pl.pallas_call(
        body,
        grid=grid,
        in_specs=[
            pl.BlockSpec((blk, d), lambda i: (i, 0)),
            pl.BlockSpec((d, d), lambda i: (0, 0)),
            pl.BlockSpec((blk, 2), lambda i: (i, 0)),
        ],
        out_specs=[
            pl.BlockSpec((blk, d), lambda i: (i, 0)),
            pl.BlockSpec((blk, 1), lambda i: (i, 0)),
        ],
        out_shape=[
            jax.ShapeDtypeStruct((n, d), jnp.float32),
            jax.ShapeDtypeStruct((n, 1), jnp.float32),
        ],
    )(x, W, degT)


def _tc_layer(a0, a1, hws, dinv, b, g, bt, Wn, n, d, blk):
    def body(a0_ref, a1_ref, hws_ref, dinv_ref, b_ref, g_ref, bt_ref, w_ref,
             h_ref, hwn_ref):
        dinv = dinv_ref[...]
        pre = (a0_ref[...] + a1_ref[...] + hws_ref[...]) * dinv + b_ref[...]
        t = pre * (g_ref[...] * _BN) + bt_ref[...]
        h = jnp.maximum(t, 0.0)
        h_ref[...] = h
        hwn_ref[...] = jnp.dot(h, w_ref[...],
                               preferred_element_type=jnp.float32) * dinv

    grid = (n // blk,)
    return pl.pallas_call(
        body,
        grid=grid,
        in_specs=[
            pl.BlockSpec((blk, d), lambda i: (i, 0)),
            pl.BlockSpec((blk, d), lambda i: (i, 0)),
            pl.BlockSpec((blk, d), lambda i: (i, 0)),
            pl.BlockSpec((blk, 1), lambda i: (i, 0)),
            pl.BlockSpec((1, d), lambda i: (0, 0)),
            pl.BlockSpec((1, d), lambda i: (0, 0)),
            pl.BlockSpec((1, d), lambda i: (0, 0)),
            pl.BlockSpec((d, d), lambda i: (0, 0)),
        ],
        out_specs=[
            pl.BlockSpec((blk, d), lambda i: (i, 0)),
            pl.BlockSpec((blk, d), lambda i: (i, 0)),
        ],
        out_shape=[
            jax.ShapeDtypeStruct((n, d), jnp.float32),
            jax.ShapeDtypeStruct((n, d), jnp.float32),
        ],
    )(a0, a1, hws, dinv, b, g, bt, Wn)


def _tc_final(a0, a1, hws, dinv, b, g, bt, x, h1, h2, Wc, bc, n, d, c, blk):
    def body(a0_ref, a1_ref, hws_ref, dinv_ref, b_ref, g_ref, bt_ref,
             x_ref, h1_ref, h2_ref, wc_ref, bc_ref, out_ref):
        dinv = dinv_ref[...]
        pre = (a0_ref[...] + a1_ref[...] + hws_ref[...]) * dinv + b_ref[...]
        t = pre * (g_ref[...] * _BN) + bt_ref[...]
        h3 = jnp.maximum(t, 0.0)
        wc = wc_ref[...]
        logits = (
            jnp.dot(x_ref[...], wc[0 * d:1 * d], preferred_element_type=jnp.float32)
            + jnp.dot(h1_ref[...], wc[1 * d:2 * d], preferred_element_type=jnp.float32)
            + jnp.dot(h2_ref[...], wc[2 * d:3 * d], preferred_element_type=jnp.float32)
            + jnp.dot(h3, wc[3 * d:4 * d], preferred_element_type=jnp.float32)
            + bc_ref[...]
        )
        m = jnp.max(logits, axis=-1, keepdims=True)
        ex = jnp.exp(logits - m)
        lse = jnp.log(jnp.sum(ex, axis=-1, keepdims=True)) + m
        out_ref[...] = logits - lse

    grid = (n // blk,)
    return pl.pallas_call(
        body,
        grid=grid,
        in_specs=[
            pl.BlockSpec((blk, d), lambda i: (i, 0)),
            pl.BlockSpec((blk, d), lambda i: (i, 0)),
            pl.BlockSpec((blk, d), lambda i: (i, 0)),
            pl.BlockSpec((blk, 1), lambda i: (i, 0)),
            pl.BlockSpec((1, d), lambda i: (0, 0)),
            pl.BlockSpec((1, d), lambda i: (0, 0)),
            pl.BlockSpec((1, d), lambda i: (0, 0)),
            pl.BlockSpec((blk, d), lambda i: (i, 0)),
            pl.BlockSpec((blk, d), lambda i: (i, 0)),
            pl.BlockSpec((blk, d), lambda i: (i, 0)),
            pl.BlockSpec((4 * d, c), lambda i: (0, 0)),
            pl.BlockSpec((1, c), lambda i: (0, 0)),
        ],
        out_specs=pl.BlockSpec((blk, c), lambda i: (i, 0)),
        out_shape=jax.ShapeDtypeStruct((n, c), jnp.float32),
    )(a0, a1, hws, dinv, b, g, bt, x, h1, h2, Wc, bc)


def kernel(x, edge_index, W1, b1, g1, bt1, W2, b2, g2, bt2,
           W3, b3, g3, bt3, Wc, bc):
    n, d = x.shape
    c = Wc.shape[1]
    blk = 1000 if n % 1000 == 0 else 8
    src = edge_index[0]
    dst = edge_index[1]
    zeros1d = jnp.zeros((n,), jnp.float32)
    zeros2d = jnp.zeros((n, d), jnp.float32)

    degp = _sc_degree(dst, zeros1d, n)
    degT = degp.T
    b1r, g1r, bt1r = b1[None, :], g1[None, :], bt1[None, :]
    b2r, g2r, bt2r = b2[None, :], g2[None, :], bt2[None, :]
    b3r, g3r, bt3r = b3[None, :], g3[None, :], bt3[None, :]
    bcr = bc[None, :]

    hw1, dinv = _tc_pre(x, W1, degT, n, d, blk)
    a1 = _sc_agg(src, dst, hw1, zeros2d, n, d)
    h1, hw2 = _tc_layer(a1[0], a1[1], hw1, dinv, b1r, g1r, bt1r, W2, n, d, blk)
    a2 = _sc_agg(src, dst, hw2, zeros2d, n, d)
    h2, hw3 = _tc_layer(a2[0], a2[1], hw2, dinv, b2r, g2r, bt2r, W3, n, d, blk)
    a3 = _sc_agg(src, dst, hw3, zeros2d, n, d)
    return _tc_final(a3[0], a3[1], hw3, dinv, b3r, g3r, bt3r,
                     x, h1, h2, Wc, bcr, n, d, c, blk)


# speedup vs baseline: 24.0681x; 2.3597x over previous
import functools

import jax
import jax.numpy as jnp
import numpy as np
from jax import lax
from jax.experimental import pallas as pl
from jax.experimental.pallas import tpu as pltpu
from jax.experimental.pallas import tpu_sc as plsc

EPS = 1e-5
NC = 2
NS = 16
NW = NC * NS


def _chunk(ept):
    for k in range(128, 0, -8):
        if ept % k == 0:
            return k
    return None


def _deg_chunk(ept):
    for k in range(128, 0, -16):
        if ept % k == 0 and (ept // k) % 5 == 0:
            return k
    return None


def _agg_chunk(ept):
    for k in range(99, 0, -1):
        if ept % k == 0 and (ept // k) % 4 == 0:
            return k
    return None


def _sc_degree(dst3, zeros1d, n):
    nw, niter, k = dst3.shape
    ngrp = niter // 5
    mesh = plsc.VectorSubcoreMesh(core_axis_name="c", subcore_axis_name="s")

    @functools.partial(
        pl.kernel, mesh=mesh,
        out_type=jax.ShapeDtypeStruct((NC, n), jnp.float32),
        scratch_types=[
            pltpu.VMEM((niter, k), jnp.int32),
            pltpu.VMEM((k,), jnp.float32),
            pltpu.VMEM_SHARED((n,), jnp.float32),
            pltpu.SemaphoreType.DMA,
        ],
    )
    def deg_kernel(dst_hbm, z_hbm, out_hbm, didx, ones_v, acc, ssem):
        cid = lax.axis_index("c")
        sid = lax.axis_index("s")
        wid = sid * NC + cid
        pltpu.sync_copy(dst_hbm.at[wid], didx)
        for j in range(k // 16):
            ones_v[pl.ds(16 * j, 16)] = jnp.ones((16,), jnp.float32)

        @pl.when(sid == 0)
        def _():
            pltpu.sync_copy(z_hbm, acc)

        plsc.subcore_barrier()

        def body(o, _):
            for b in range(5):
                pltpu.async_copy(ones_v, acc.at[didx.at[o * 5 + b]], ssem,
                                 add=True)
            for b in range(5):
                pltpu.make_async_copy(ones_v, acc.at[didx.at[o * 5 + b]],
                                      ssem).wait()
            return ()

        lax.fori_loop(0, ngrp, body, (), unroll=False)
        plsc.subcore_barrier()

        @pl.when(sid == 0)
        def _():
            pltpu.sync_copy(acc, out_hbm.at[cid])

    return deg_kernel(dst3, zeros1d)


def _sc_agg(src3, dst3, hw, zeros2d, n, d):
    nw, niter, k = src3.shape
    nouter = niter // 4
    rpt = (n // NS) // 8 * 8
    tail = n - NS * rpt
    mesh = plsc.VectorSubcoreMesh(core_axis_name="c", subcore_axis_name="s")

    @functools.partial(
        pl.kernel, mesh=mesh,
        out_type=jax.ShapeDtypeStruct((NC, n, d), jnp.float32),
        scratch_types=[
            pltpu.VMEM((4, k), jnp.int32),
            pltpu.VMEM((4, k), jnp.int32),
            pltpu.VMEM((4, k, d), jnp.float32),
            pltpu.VMEM_SHARED((n, d), jnp.float32),
            [pltpu.SemaphoreType.DMA] * 4,
            pltpu.SemaphoreType.DMA,
        ],
    )
    def agg_kernel(src_hbm, dst_hbm, hw_hbm, z_hbm, out_hbm,
                   sidx, didx, rows, acc, sems, ssem):
        cid = lax.axis_index("c")
        sid = lax.axis_index("s")
        wid = sid * NC + cid

        def idx_load(i, b):
            pltpu.async_copy(src_hbm.at[wid, i], sidx.at[b], sems[b])
            pltpu.async_copy(dst_hbm.at[wid, i], didx.at[b], sems[b])

        def idx_wait_and_gather(i, b):
            pltpu.make_async_copy(src_hbm.at[wid, i], sidx.at[b],
                                  sems[b]).wait()
            pltpu.make_async_copy(dst_hbm.at[wid, i], didx.at[b],
                                  sems[b]).wait()
            pltpu.async_copy(hw_hbm.at[sidx.at[b]], rows.at[b], sems[b])

        for j in range(min(3, niter)):
            idx_load(j, j)
        for j in range(min(2, niter)):
            idx_wait_and_gather(j, j)

        pltpu.sync_copy(z_hbm.at[pl.ds(sid * rpt, rpt)],
                        acc.at[pl.ds(sid * rpt, rpt)])
        if tail:
            @pl.when(sid == NS - 1)
            def _():
                pltpu.sync_copy(z_hbm.at[pl.ds(NS * rpt, tail)],
                                acc.at[pl.ds(NS * rpt, tail)])
        plsc.subcore_barrier()

        def outer(o, _):
            for b in range(4):
                i = o * 4 + b

                def _wait_prev(pb=(b - 1) % 4):
                    pltpu.make_async_copy(rows.at[pb], acc.at[didx.at[pb]],
                                          ssem).wait()
                if b == 0:
                    pl.when(o >= 1)(_wait_prev)
                else:
                    _wait_prev()

                def _pref(ni=i + 3, nb=(b + 3) % 4):
                    idx_load(ni, nb)
                if b == 0:
                    _pref()
                else:
                    pl.when(o < nouter - 1)(_pref)

                def _gath(gi=i + 2, gb=(b + 2) % 4):
                    idx_wait_and_gather(gi, gb)
                if b <= 1:
                    _gath()
                else:
                    pl.when(o < nouter - 1)(_gath)

                pltpu.make_async_copy(hw_hbm.at[sidx.at[b]], rows.at[b],
                                      sems[b]).wait()
                pltpu.async_copy(rows.at[b], acc.at[didx.at[b]], ssem,
                                 add=True)
            return ()

        lax.fori_loop(0, nouter, outer, (), unroll=False)
        pltpu.make_async_copy(rows.at[3], acc.at[didx.at[3]], ssem).wait()
        plsc.subcore_barrier()
        pltpu.sync_copy(acc.at[pl.ds(sid * rpt, rpt)],
                        out_hbm.at[cid, pl.ds(sid * rpt, rpt)])
        if tail:
            @pl.when(sid == NS - 1)
            def _():
                pltpu.sync_copy(acc.at[pl.ds(NS * rpt, tail)],
                                out_hbm.at[cid, pl.ds(NS * rpt, tail)])

    return agg_kernel(src3, dst3, hw, zeros2d)


_BN = float(1.0 / np.sqrt(1.0 + EPS))


def _tc_pre(x, W, degT, n, d, blk):
    def body(x_ref, w_ref, deg_ref, hw_ref, dinv_ref):
        dg = deg_ref[:, 0:1] + deg_ref[:, 1:2] + 1.0
        dinv = lax.rsqrt(dg)
        dinv_ref[...] = dinv
        hw_ref[...] = jnp.dot(x_ref[...], w_ref[...],
                              preferred_element_type=jnp.float32) * dinv

    grid = (n // blk,)
    return pl.pallas_call(
        body,
        grid=grid,
        in_specs=[
            pl.BlockSpec((blk, d), lambda i: (i, 0)),
            pl.BlockSpec((d, d), lambda i: (0, 0)),
            pl.BlockSpec((blk, 2), lambda i: (i, 0)),
        ],
        out_specs=[
            pl.BlockSpec((blk, d), lambda i: (i, 0)),
            pl.BlockSpec((blk, 1), lambda i: (i, 0)),
        ],
        out_shape=[
            jax.ShapeDtypeStruct((n, d), jnp.float32),
            jax.ShapeDtypeStruct((n, 1), jnp.float32),
        ],
    )(x, W, degT)


def _tc_layer(a0, a1, hws, dinv, b, g, bt, Wn, n, d, blk):
    def body(a0_ref, a1_ref, hws_ref, dinv_ref, b_ref, g_ref, bt_ref, w_ref,
             h_ref, hwn_ref):
        dinv = dinv_ref[...]
        pre = (a0_ref[...] + a1_ref[...] + hws_ref[...]) * dinv + b_ref[...]
        t = pre * (g_ref[...] * _BN) + bt_ref[...]
        h = jnp.maximum(t, 0.0)
        h_ref[...] = h
        hwn_ref[...] = jnp.dot(h, w_ref[...],
                               preferred_element_type=jnp.float32) * dinv

    grid = (n // blk,)
    return pl.pallas_call(
        body,
        grid=grid,
        in_specs=[
            pl.BlockSpec((blk, d), lambda i: (i, 0)),
            pl.BlockSpec((blk, d), lambda i: (i, 0)),
            pl.BlockSpec((blk, d), lambda i: (i, 0)),
            pl.BlockSpec((blk, 1), lambda i: (i, 0)),
            pl.BlockSpec((1, d), lambda i: (0, 0)),
            pl.BlockSpec((1, d), lambda i: (0, 0)),
            pl.BlockSpec((1, d), lambda i: (0, 0)),
            pl.BlockSpec((d, d), lambda i: (0, 0)),
        ],
        out_specs=[
            pl.BlockSpec((blk, d), lambda i: (i, 0)),
            pl.BlockSpec((blk, d), lambda i: (i, 0)),
        ],
        out_shape=[
            jax.ShapeDtypeStruct((n, d), jnp.float32),
            jax.ShapeDtypeStruct((n, d), jnp.float32),
        ],
    )(a0, a1, hws, dinv, b, g, bt, Wn)


def _tc_final(a0, a1, hws, dinv, b, g, bt, x, h1, h2, Wc, bc, n, d, c, blk):
    def body(a0_ref, a1_ref, hws_ref, dinv_ref, b_ref, g_ref, bt_ref,
             x_ref, h1_ref, h2_ref, wc_ref, bc_ref, out_ref):
        dinv = dinv_ref[...]
        pre = (a0_ref[...] + a1_ref[...] + hws_ref[...]) * dinv + b_ref[...]
        t = pre * (g_ref[...] * _BN) + bt_ref[...]
        h3 = jnp.maximum(t, 0.0)
        wc = wc_ref[...]
        logits = (
            jnp.dot(x_ref[...], wc[0 * d:1 * d], preferred_element_type=jnp.float32)
            + jnp.dot(h1_ref[...], wc[1 * d:2 * d], preferred_element_type=jnp.float32)
            + jnp.dot(h2_ref[...], wc[2 * d:3 * d], preferred_element_type=jnp.float32)
            + jnp.dot(h3, wc[3 * d:4 * d], preferred_element_type=jnp.float32)
            + bc_ref[...]
        )
        m = jnp.max(logits, axis=-1, keepdims=True)
        ex = jnp.exp(logits - m)
        lse = jnp.log(jnp.sum(ex, axis=-1, keepdims=True)) + m
        out_ref[...] = logits - lse

    grid = (n // blk,)
    return pl.pallas_call(
        body,
        grid=grid,
        in_specs=[
            pl.BlockSpec((blk, d), lambda i: (i, 0)),
            pl.BlockSpec((blk, d), lambda i: (i, 0)),
            pl.BlockSpec((blk, d), lambda i: (i, 0)),
            pl.BlockSpec((blk, 1), lambda i: (i, 0)),
            pl.BlockSpec((1, d), lambda i: (0, 0)),
            pl.BlockSpec((1, d), lambda i: (0, 0)),
            pl.BlockSpec((1, d), lambda i: (0, 0)),
            pl.BlockSpec((blk, d), lambda i: (i, 0)),
            pl.BlockSpec((blk, d), lambda i: (i, 0)),
            pl.BlockSpec((blk, d), lambda i: (i, 0)),
            pl.BlockSpec((4 * d, c), lambda i: (0, 0)),
            pl.BlockSpec((1, c), lambda i: (0, 0)),
        ],
        out_specs=pl.BlockSpec((blk, c), lambda i: (i, 0)),
        out_shape=jax.ShapeDtypeStruct((n, c), jnp.float32),
    )(a0, a1, hws, dinv, b, g, bt, x, h1, h2, Wc, bc)


def kernel(x, edge_index, W1, b1, g1, bt1, W2, b2, g2, bt2,
           W3, b3, g3, bt3, Wc, bc):
    n, d = x.shape
    c = Wc.shape[1]
    blk = 1000 if n % 1000 == 0 else 8
    src = edge_index[0]
    dst = edge_index[1]
    e = src.shape[0]
    ept = e // NW
    dk = _deg_chunk(ept)
    ak = _agg_chunk(ept)
    dst3d = dst.reshape(NW, ept // dk, dk)
    srcA = src.reshape(NW, ept // ak, ak)
    dstA = dst.reshape(NW, ept // ak, ak)
    zeros1d = jnp.zeros((n,), jnp.float32)
    zeros2d = jnp.zeros((n, d), jnp.float32)

    degp = _sc_degree(dst3d, zeros1d, n)
    degT = degp.T
    b1r, g1r, bt1r = b1[None, :], g1[None, :], bt1[None, :]
    b2r, g2r, bt2r = b2[None, :], g2[None, :], bt2[None, :]
    b3r, g3r, bt3r = b3[None, :], g3[None, :], bt3[None, :]
    bcr = bc[None, :]

    hw1, dinv = _tc_pre(x, W1, degT, n, d, blk)
    a1 = _sc_agg(srcA, dstA, hw1, zeros2d, n, d)
    h1, hw2 = _tc_layer(a1[0], a1[1], hw1, dinv, b1r, g1r, bt1r, W2, n, d, blk)
    a2 = _sc_agg(srcA, dstA, hw2, zeros2d, n, d)
    h2, hw3 = _tc_layer(a2[0], a2[1], hw2, dinv, b2r, g2r, bt2r, W3, n, d, blk)
    a3 = _sc_agg(srcA, dstA, hw3, zeros2d, n, d)
    return _tc_final(a3[0], a3[1], hw3, dinv, b3r, g3r, bt3r,
                     x, h1, h2, Wc, bcr, n, d, c, blk)


# speedup vs baseline: 25.3500x; 1.0533x over previous
import functools

import jax
import jax.numpy as jnp
import numpy as np
from jax import lax
from jax.experimental import pallas as pl
from jax.experimental.pallas import tpu as pltpu
from jax.experimental.pallas import tpu_sc as plsc

EPS = 1e-5
NC = 2
NS = 16
NW = NC * NS


def _chunk(ept):
    for k in range(128, 0, -8):
        if ept % k == 0:
            return k
    return None


def _deg_chunk(ept):
    for k in range(128, 0, -16):
        if ept % k == 0 and (ept // k) % 5 == 0:
            return k
    return None


def _agg_chunk(ept):
    for k in range(99, 0, -1):
        if ept % k == 0 and (ept // k) % 8 == 0:
            return k
    return None


def _sc_degree(dst3, zeros1d, n):
    nw, niter, k = dst3.shape
    ngrp = niter // 5
    mesh = plsc.VectorSubcoreMesh(core_axis_name="c", subcore_axis_name="s")

    @functools.partial(
        pl.kernel, mesh=mesh,
        out_type=jax.ShapeDtypeStruct((NC, n), jnp.float32),
        scratch_types=[
            pltpu.VMEM((niter, k), jnp.int32),
            pltpu.VMEM((k,), jnp.float32),
            pltpu.VMEM_SHARED((n,), jnp.float32),
            pltpu.SemaphoreType.DMA,
        ],
    )
    def deg_kernel(dst_hbm, z_hbm, out_hbm, didx, ones_v, acc, ssem):
        cid = lax.axis_index("c")
        sid = lax.axis_index("s")
        wid = sid * NC + cid
        pltpu.sync_copy(dst_hbm.at[wid], didx)
        for j in range(k // 16):
            ones_v[pl.ds(16 * j, 16)] = jnp.ones((16,), jnp.float32)

        @pl.when(sid == 0)
        def _():
            pltpu.sync_copy(z_hbm, acc)

        plsc.subcore_barrier()

        def body(o, _):
            for b in range(5):
                pltpu.async_copy(ones_v, acc.at[didx.at[o * 5 + b]], ssem,
                                 add=True)
            for b in range(5):
                pltpu.make_async_copy(ones_v, acc.at[didx.at[o * 5 + b]],
                                      ssem).wait()
            return ()

        lax.fori_loop(0, ngrp, body, (), unroll=False)
        plsc.subcore_barrier()

        @pl.when(sid == 0)
        def _():
            pltpu.sync_copy(acc, out_hbm.at[cid])

    return deg_kernel(dst3, zeros1d)


def _sc_agg(src3, dst3, hw, zeros2d, n, d):
    nw, niter, k = src3.shape
    nouter = niter // 8
    rpt = (n // NS) // 8 * 8
    tail = n - NS * rpt
    mesh = plsc.VectorSubcoreMesh(core_axis_name="c", subcore_axis_name="s")

    @functools.partial(
        pl.kernel, mesh=mesh,
        out_type=jax.ShapeDtypeStruct((NC, n, d), jnp.float32),
        scratch_types=[
            pltpu.VMEM((8, k), jnp.int32),
            pltpu.VMEM((8, k), jnp.int32),
            pltpu.VMEM((4, k, d), jnp.float32),
            pltpu.VMEM_SHARED((n, d), jnp.float32),
            [pltpu.SemaphoreType.DMA] * 4,
            pltpu.SemaphoreType.DMA,
        ],
    )
    def agg_kernel(src_hbm, dst_hbm, hw_hbm, z_hbm, out_hbm,
                   sidx, didx, rows, acc, sems, ssem):
        cid = lax.axis_index("c")
        sid = lax.axis_index("s")
        wid = sid * NC + cid

        def idx_load(i, ib):
            pltpu.async_copy(src_hbm.at[wid, i], sidx.at[ib], sems[ib % 4])
            pltpu.async_copy(dst_hbm.at[wid, i], didx.at[ib], sems[ib % 4])

        def idx_wait_and_gather(i, ib):
            pltpu.make_async_copy(src_hbm.at[wid, i], sidx.at[ib],
                                  sems[ib % 4]).wait()
            pltpu.make_async_copy(dst_hbm.at[wid, i], didx.at[ib],
                                  sems[ib % 4]).wait()
            pltpu.async_copy(hw_hbm.at[sidx.at[ib]], rows.at[ib % 4],
                             sems[ib % 4])

        for j in range(min(3, niter)):
            idx_load(j, j)
        for j in range(min(2, niter)):
            idx_wait_and_gather(j, j)

        pltpu.sync_copy(z_hbm.at[pl.ds(sid * rpt, rpt)],
                        acc.at[pl.ds(sid * rpt, rpt)])
        if tail:
            @pl.when(sid == NS - 1)
            def _():
                pltpu.sync_copy(z_hbm.at[pl.ds(NS * rpt, tail)],
                                acc.at[pl.ds(NS * rpt, tail)])
        plsc.subcore_barrier()

        def outer(o, _):
            for b in range(8):
                i = o * 8 + b

                def _wait_prev(pb=(b - 2) % 4, pib=(b - 2) % 8):
                    pltpu.make_async_copy(rows.at[pb], acc.at[didx.at[pib]],
                                          ssem).wait()
                if b <= 1:
                    pl.when(o >= 1)(_wait_prev)
                else:
                    _wait_prev()

                def _pref(ni=i + 3, nb=(b + 3) % 8):
                    idx_load(ni, nb)
                if b <= 4:
                    _pref()
                else:
                    pl.when(o < nouter - 1)(_pref)

                def _gath(gi=i + 2, gb=(b + 2) % 8):
                    idx_wait_and_gather(gi, gb)
                if b <= 5:
                    _gath()
                else:
                    pl.when(o < nouter - 1)(_gath)

                pltpu.make_async_copy(hw_hbm.at[sidx.at[b]], rows.at[b % 4],
                                      sems[b % 4]).wait()
                pltpu.async_copy(rows.at[b % 4], acc.at[didx.at[b]], ssem,
                                 add=True)
            return ()

        lax.fori_loop(0, nouter, outer, (), unroll=False)
        pltpu.make_async_copy(rows.at[2], acc.at[didx.at[6]], ssem).wait()
        pltpu.make_async_copy(rows.at[3], acc.at[didx.at[7]], ssem).wait()
        plsc.subcore_barrier()
        pltpu.sync_copy(acc.at[pl.ds(sid * rpt, rpt)],
                        out_hbm.at[cid, pl.ds(sid * rpt, rpt)])
        if tail:
            @pl.when(sid == NS - 1)
            def _():
                pltpu.sync_copy(acc.at[pl.ds(NS * rpt, tail)],
                                out_hbm.at[cid, pl.ds(NS * rpt, tail)])

    return agg_kernel(src3, dst3, hw, zeros2d)


_BN = float(1.0 / np.sqrt(1.0 + EPS))


def _tc_pre(x, W, degT, n, d, blk):
    def body(x_ref, w_ref, deg_ref, hw_ref, dinv_ref):
        dg = deg_ref[:, 0:1] + deg_ref[:, 1:2] + 1.0
        dinv = lax.rsqrt(dg)
        dinv_ref[...] = dinv
        hw_ref[...] = jnp.dot(x_ref[...], w_ref[...],
                              preferred_element_type=jnp.float32) * dinv

    grid = (n // blk,)
    return pl.pallas_call(
        body,
        grid=grid,
        in_specs=[
            pl.BlockSpec((blk, d), lambda i: (i, 0)),
            pl.BlockSpec((d, d), lambda i: (0, 0)),
            pl.BlockSpec((blk, 2), lambda i: (i, 0)),
        ],
        out_specs=[
            pl.BlockSpec((blk, d), lambda i: (i, 0)),
            pl.BlockSpec((blk, 1), lambda i: (i, 0)),
        ],
        out_shape=[
            jax.ShapeDtypeStruct((n, d), jnp.float32),
            jax.ShapeDtypeStruct((n, 1), jnp.float32),
        ],
    )(x, W, degT)


def _tc_layer(a0, a1, hws, dinv, b, g, bt, Wn, n, d, blk):
    def body(a0_ref, a1_ref, hws_ref, dinv_ref, b_ref, g_ref, bt_ref, w_ref,
             h_ref, hwn_ref):
        dinv = dinv_ref[...]
        pre = (a0_ref[...] + a1_ref[...] + hws_ref[...]) * dinv + b_ref[...]
        t = pre * (g_ref[...] * _BN) + bt_ref[...]
        h = jnp.maximum(t, 0.0)
        h_ref[...] = h
        hwn_ref[...] = jnp.dot(h, w_ref[...],
                               preferred_element_type=jnp.float32) * dinv

    grid = (n // blk,)
    return pl.pallas_call(
        body,
        grid=grid,
        in_specs=[
            pl.BlockSpec((blk, d), lambda i: (i, 0)),
            pl.BlockSpec((blk, d), lambda i: (i, 0)),
            pl.BlockSpec((blk, d), lambda i: (i, 0)),
            pl.BlockSpec((blk, 1), lambda i: (i, 0)),
            pl.BlockSpec((1, d), lambda i: (0, 0)),
            pl.BlockSpec((1, d), lambda i: (0, 0)),
            pl.BlockSpec((1, d), lambda i: (0, 0)),
            pl.BlockSpec((d, d), lambda i: (0, 0)),
        ],
        out_specs=[
            pl.BlockSpec((blk, d), lambda i: (i, 0)),
            pl.BlockSpec((blk, d), lambda i: (i, 0)),
        ],
        out_shape=[
            jax.ShapeDtypeStruct((n, d), jnp.float32),
            jax.ShapeDtypeStruct((n, d), jnp.float32),
        ],
    )(a0, a1, hws, dinv, b, g, bt, Wn)


def _tc_final(a0, a1, hws, dinv, b, g, bt, x, h1, h2, Wc, bc, n, d, c, blk):
    def body(a0_ref, a1_ref, hws_ref, dinv_ref, b_ref, g_ref, bt_ref,
             x_ref, h1_ref, h2_ref, wc_ref, bc_ref, out_ref):
        dinv = dinv_ref[...]
        pre = (a0_ref[...] + a1_ref[...] + hws_ref[...]) * dinv + b_ref[...]
        t = pre * (g_ref[...] * _BN) + bt_ref[...]
        h3 = jnp.maximum(t, 0.0)
        wc = wc_ref[...]
        logits = (
            jnp.dot(x_ref[...], wc[0 * d:1 * d], preferred_element_type=jnp.float32)
            + jnp.dot(h1_ref[...], wc[1 * d:2 * d], preferred_element_type=jnp.float32)
            + jnp.dot(h2_ref[...], wc[2 * d:3 * d], preferred_element_type=jnp.float32)
            + jnp.dot(h3, wc[3 * d:4 * d], preferred_element_type=jnp.float32)
            + bc_ref[...]
        )
        m = jnp.max(logits, axis=-1, keepdims=True)
        ex = jnp.exp(logits - m)
        lse = jnp.log(jnp.sum(ex, axis=-1, keepdims=True)) + m
        out_ref[...] = logits - lse

    grid = (n // blk,)
    return pl.pallas_call(
        body,
        grid=grid,
        in_specs=[
            pl.BlockSpec((blk, d), lambda i: (i, 0)),
            pl.BlockSpec((blk, d), lambda i: (i, 0)),
            pl.BlockSpec((blk, d), lambda i: (i, 0)),
            pl.BlockSpec((blk, 1), lambda i: (i, 0)),
            pl.BlockSpec((1, d), lambda i: (0, 0)),
            pl.BlockSpec((1, d), lambda i: (0, 0)),
            pl.BlockSpec((1, d), lambda i: (0, 0)),
            pl.BlockSpec((blk, d), lambda i: (i, 0)),
            pl.BlockSpec((blk, d), lambda i: (i, 0)),
            pl.BlockSpec((blk, d), lambda i: (i, 0)),
            pl.BlockSpec((4 * d, c), lambda i: (0, 0)),
            pl.BlockSpec((1, c), lambda i: (0, 0)),
        ],
        out_specs=pl.BlockSpec((blk, c), lambda i: (i, 0)),
        out_shape=jax.ShapeDtypeStruct((n, c), jnp.float32),
    )(a0, a1, hws, dinv, b, g, bt, x, h1, h2, Wc, bc)


def kernel(x, edge_index, W1, b1, g1, bt1, W2, b2, g2, bt2,
           W3, b3, g3, bt3, Wc, bc):
    n, d = x.shape
    c = Wc.shape[1]
    blk = 1000 if n % 1000 == 0 else 8
    src = edge_index[0]
    dst = edge_index[1]
    e = src.shape[0]
    ept = e // NW
    dk = _deg_chunk(ept)
    ak = _agg_chunk(ept)
    dst3d = dst.reshape(NW, ept // dk, dk)
    srcA = src.reshape(NW, ept // ak, ak)
    dstA = dst.reshape(NW, ept // ak, ak)
    zeros1d = jnp.zeros((n,), jnp.float32)
    zeros2d = jnp.zeros((n, d), jnp.float32)

    degp = _sc_degree(dst3d, zeros1d, n)
    degT = degp.T
    b1r, g1r, bt1r = b1[None, :], g1[None, :], bt1[None, :]
    b2r, g2r, bt2r = b2[None, :], g2[None, :], bt2[None, :]
    b3r, g3r, bt3r = b3[None, :], g3[None, :], bt3[None, :]
    bcr = bc[None, :]

    hw1, dinv = _tc_pre(x, W1, degT, n, d, blk)
    a1 = _sc_agg(srcA, dstA, hw1, zeros2d, n, d)
    h1, hw2 = _tc_layer(a1[0], a1[1], hw1, dinv, b1r, g1r, bt1r, W2, n, d, blk)
    a2 = _sc_agg(srcA, dstA, hw2, zeros2d, n, d)
    h2, hw3 = _tc_layer(a2[0], a2[1], hw2, dinv, b2r, g2r, bt2r, W3, n, d, blk)
    a3 = _sc_agg(srcA, dstA, hw3, zeros2d, n, d)
    return _tc_final(a3[0], a3[1], hw3, dinv, b3r, g3r, bt3r,
                     x, h1, h2, Wc, bcr, n, d, c, blk)
